# baseline (speedup 1.0000x reference)
import functools

import jax
import jax.numpy as jnp
from jax.experimental import pallas as pl
from jax.experimental.pallas import tpu as pltpu

SCORE_THRESH = 0.05
NMS_THRESH = 0.5
PRE_NMS_TOPK = 1000
DET_PER_IMG = 100
IOU_MATCH_THRESH = 0.5

_N = 20000
_NPAD = 20480
_ROWS = 160
_BM = 16
_G = 100
_C = 1024


_GP = 104


def _match_kernel(gt_ref, x1_ref, y1_ref, x2_ref, y2_ref, midx_ref, mlab_ref):
    gx1 = gt_ref[:, 0:1]
    gy1 = gt_ref[:, 1:2]
    gx2 = gt_ref[:, 2:3]
    gy2 = gt_ref[:, 3:4]
    ga = (gx2 - gx1) * (gy2 - gy1)
    giota = jax.lax.broadcasted_iota(jnp.int32, (_GP, 128), 0)

    def row(r, _):
        x1 = x1_ref[pl.ds(r, 1), :]
        y1 = y1_ref[pl.ds(r, 1), :]
        x2 = x2_ref[pl.ds(r, 1), :]
        y2 = y2_ref[pl.ds(r, 1), :]
        area = (x2 - x1) * (y2 - y1)
        ix1 = jnp.maximum(gx1, x1)
        iy1 = jnp.maximum(gy1, y1)
        ix2 = jnp.minimum(gx2, x2)
        iy2 = jnp.minimum(gy2, y2)
        w = jnp.maximum(ix2 - ix1, 0.0)
        h = jnp.maximum(iy2 - iy1, 0.0)
        inter = w * h
        iou = inter / (ga + area - inter + 1e-9)
        mv = jnp.max(iou, axis=0, keepdims=True)
        cand = jnp.where(iou == mv, giota, _GP)
        mi = jnp.min(cand, axis=0, keepdims=True)
        midx_ref[pl.ds(r, 1), :] = mi
        mlab_ref[pl.ds(r, 1), :] = (mv >= IOU_MATCH_THRESH).astype(jnp.int32)
        return 0

    jax.lax.fori_loop(0, _BM, row, 0)


def _nms_kernel(cand_ref, candt_ref, srow_ref, kept_ref, iou_ref, keep_ref):
    x1r = candt_ref[0:1, :]
    y1r = candt_ref[1:2, :]
    x2r = candt_ref[2:3, :]
    y2r = candt_ref[3:4, :]
    arear = (x2r - x1r) * (y2r - y1r)

    rb = 8

    def build(r, _):
        s = r * rb
        x1c = cand_ref[pl.ds(s, rb), 0:1]
        y1c = cand_ref[pl.ds(s, rb), 1:2]
        x2c = cand_ref[pl.ds(s, rb), 2:3]
        y2c = cand_ref[pl.ds(s, rb), 3:4]
        areac = (x2c - x1c) * (y2c - y1c)
        ix1 = jnp.maximum(x1c, x1r)
        iy1 = jnp.maximum(y1c, y1r)
        ix2 = jnp.minimum(x2c, x2r)
        iy2 = jnp.minimum(y2c, y2r)
        w = jnp.maximum(ix2 - ix1, 0.0)
        h = jnp.maximum(iy2 - iy1, 0.0)
        inter = w * h
        iou_ref[pl.ds(s, rb), :] = inter / (areac + arear - inter + 1e-9)
        return 0

    jax.lax.fori_loop(0, _C // rb, build, 0)

    keep_ref[...] = jnp.ones((8, _C), dtype=jnp.float32)

    ciota = jax.lax.broadcasted_iota(jnp.int32, (1, 128), 1)
    iota_full = jax.lax.broadcasted_iota(jnp.int32, (1, _C), 1)
    for k in range(_C // 128):
        base = k * 128

        def inner(i, _, base=base):
            gi = base + i
            chunk = keep_ref[0:1, pl.ds(base, 128)]
            keep_i = jnp.max(jnp.where(ciota == i, chunk, 0.0),
                             axis=1, keepdims=True)[0, 0]

            @pl.when(keep_i > 0.5)
            def _():
                row = iou_ref[pl.ds(gi, 1), :]
                cur = keep_ref[0:1, :]
                sup = (row > NMS_THRESH) & (iota_full > gi)
                keep_ref[0:1, :] = jnp.where(sup, 0.0, cur)

            return 0

        jax.lax.fori_loop(0, 128, inner, 0)

    kept = jnp.where(keep_ref[0:1, :] > 0.5, srow_ref[0:1, :], -1e30)
    kept_ref[...] = jnp.broadcast_to(kept, (8, _C))


@jax.jit
def kernel(boxes, scores, gt_boxes):
    bx = jnp.pad(boxes, ((0, _NPAD - _N), (0, 0)))
    comps = [bx[:, k].reshape(_ROWS, 128) for k in range(4)]
    blk = pl.BlockSpec((_BM, 128), lambda i: (i, 0))
    midx, mlab = pl.pallas_call(
        _match_kernel,
        grid=(_ROWS // _BM,),
        in_specs=[pl.BlockSpec((_GP, 4), lambda i: (0, 0))] + [blk] * 4,
        out_specs=[blk, blk],
        out_shape=[
            jax.ShapeDtypeStruct((_ROWS, 128), jnp.int32),
            jax.ShapeDtypeStruct((_ROWS, 128), jnp.int32),
        ],
    )(jnp.pad(gt_boxes, ((0, _GP - _G), (0, 0))), *comps)
    matched_idxs = midx.reshape(-1)[:_N]
    match_labels = mlab.reshape(-1)[:_N]

    scores_f = jnp.where(scores > SCORE_THRESH, scores, -1e30)
    cand_scores, cand_idx = jax.lax.top_k(scores_f, PRE_NMS_TOPK)
    cand_boxes = boxes[cand_idx]
    candp = jnp.pad(cand_boxes, ((0, _C - PRE_NMS_TOPK), (0, 0)))
    candt = jnp.pad(candp.T, ((0, 4), (0, 0)))
    srow = jnp.full((8, _C), -1e30, dtype=jnp.float32)
    srow = srow.at[0, :PRE_NMS_TOPK].set(cand_scores)

    kept = pl.pallas_call(
        _nms_kernel,
        out_shape=jax.ShapeDtypeStruct((8, _C), jnp.float32),
        scratch_shapes=[pltpu.VMEM((_C, _C), jnp.float32),
                        pltpu.VMEM((8, _C), jnp.float32)],
    )(candp, candt, srow)
    kept_scores = kept[0, :PRE_NMS_TOPK]

    det_scores, det_idx = jax.lax.top_k(kept_scores, DET_PER_IMG)
    valid = det_scores > -1e29
    det_boxes = jnp.where(valid[:, None], cand_boxes[det_idx], 0.0)
    det_scores = jnp.where(valid, det_scores, 0.0)
    det = jnp.concatenate([det_boxes, det_scores[:, None]], axis=1)
    return det, matched_idxs, match_labels


# speedup vs baseline: 1.2290x; 1.2290x over previous
import functools

import jax
import jax.numpy as jnp
from jax.experimental import pallas as pl
from jax.experimental.pallas import tpu as pltpu

SCORE_THRESH = 0.05
NMS_THRESH = 0.5
PRE_NMS_TOPK = 1000
DET_PER_IMG = 100
IOU_MATCH_THRESH = 0.5

_N = 20000
_NPAD = 20480
_ROWS = 160
_BM = 16
_G = 100
_C = 1024


_GP = 104


def _match_kernel(gt_ref, x1_ref, y1_ref, x2_ref, y2_ref, midx_ref, mlab_ref):
    gx1 = gt_ref[:, 0:1]
    gy1 = gt_ref[:, 1:2]
    gx2 = gt_ref[:, 2:3]
    gy2 = gt_ref[:, 3:4]
    ga = (gx2 - gx1) * (gy2 - gy1)
    giota = jax.lax.broadcasted_iota(jnp.int32, (_GP, 128), 0)

    def row(r, _):
        x1 = x1_ref[pl.ds(r, 1), :]
        y1 = y1_ref[pl.ds(r, 1), :]
        x2 = x2_ref[pl.ds(r, 1), :]
        y2 = y2_ref[pl.ds(r, 1), :]
        area = (x2 - x1) * (y2 - y1)
        ix1 = jnp.maximum(gx1, x1)
        iy1 = jnp.maximum(gy1, y1)
        ix2 = jnp.minimum(gx2, x2)
        iy2 = jnp.minimum(gy2, y2)
        w = jnp.maximum(ix2 - ix1, 0.0)
        h = jnp.maximum(iy2 - iy1, 0.0)
        inter = w * h
        iou = inter / (ga + area - inter + 1e-9)
        mv = jnp.max(iou, axis=0, keepdims=True)
        cand = jnp.where(iou == mv, giota, _GP)
        mi = jnp.min(cand, axis=0, keepdims=True)
        midx_ref[pl.ds(r, 1), :] = mi
        mlab_ref[pl.ds(r, 1), :] = (mv >= IOU_MATCH_THRESH).astype(jnp.int32)
        return 0

    jax.lax.fori_loop(0, _BM, row, 0)


def _nms_kernel(cand_ref, candt_ref, srow_ref, kept_ref, iou_ref):
    x1r = candt_ref[0:1, :]
    y1r = candt_ref[1:2, :]
    x2r = candt_ref[2:3, :]
    y2r = candt_ref[3:4, :]
    arear = (x2r - x1r) * (y2r - y1r)

    rb = 8

    def build(r, _):
        s = r * rb
        x1c = cand_ref[pl.ds(s, rb), 0:1]
        y1c = cand_ref[pl.ds(s, rb), 1:2]
        x2c = cand_ref[pl.ds(s, rb), 2:3]
        y2c = cand_ref[pl.ds(s, rb), 3:4]
        areac = (x2c - x1c) * (y2c - y1c)
        ix1 = jnp.maximum(x1c, x1r)
        iy1 = jnp.maximum(y1c, y1r)
        ix2 = jnp.minimum(x2c, x2r)
        iy2 = jnp.minimum(y2c, y2r)
        w = jnp.maximum(ix2 - ix1, 0.0)
        h = jnp.maximum(iy2 - iy1, 0.0)
        inter = w * h
        iou = inter / (areac + arear - inter + 1e-9)
        riota = jax.lax.broadcasted_iota(jnp.int32, (rb, _C), 0) + s
        liota = jax.lax.broadcasted_iota(jnp.int32, (rb, _C), 1)
        iou_ref[pl.ds(s, rb), :] = jnp.where(
            (iou > NMS_THRESH) & (liota > riota), 1.0, 0.0)
        return 0

    jax.lax.fori_loop(0, _C // rb, build, 0)

    ciota = jax.lax.broadcasted_iota(jnp.int32, (1, 128), 1)

    def chunk_loop(base):
        def inner(i, keep):
            gi = base + i
            chunk = jax.lax.slice(keep, (0, base), (1, base + 128))
            keep_i = jnp.max(jnp.where(ciota == i, chunk, 0.0),
                             axis=1, keepdims=True)
            sup = iou_ref[pl.ds(gi, 1), :] * keep_i
            return keep * (1.0 - sup)
        return inner

    keep = jnp.ones((1, _C), dtype=jnp.float32)
    for k in range(_C // 128):
        keep = jax.lax.fori_loop(0, 128, chunk_loop(k * 128), keep)

    kept = jnp.where(keep > 0.5, srow_ref[0:1, :], -1e30)
    kept_ref[...] = jnp.broadcast_to(kept, (8, _C))


@jax.jit
def kernel(boxes, scores, gt_boxes):
    bx = jnp.pad(boxes, ((0, _NPAD - _N), (0, 0)))
    comps = [bx[:, k].reshape(_ROWS, 128) for k in range(4)]
    blk = pl.BlockSpec((_BM, 128), lambda i: (i, 0))
    midx, mlab = pl.pallas_call(
        _match_kernel,
        grid=(_ROWS // _BM,),
        in_specs=[pl.BlockSpec((_GP, 4), lambda i: (0, 0))] + [blk] * 4,
        out_specs=[blk, blk],
        out_shape=[
            jax.ShapeDtypeStruct((_ROWS, 128), jnp.int32),
            jax.ShapeDtypeStruct((_ROWS, 128), jnp.int32),
        ],
    )(jnp.pad(gt_boxes, ((0, _GP - _G), (0, 0))), *comps)
    matched_idxs = midx.reshape(-1)[:_N]
    match_labels = mlab.reshape(-1)[:_N]

    scores_f = jnp.where(scores > SCORE_THRESH, scores, -1e30)
    cand_scores, cand_idx = jax.lax.top_k(scores_f, PRE_NMS_TOPK)
    cand_boxes = boxes[cand_idx]
    candp = jnp.pad(cand_boxes, ((0, _C - PRE_NMS_TOPK), (0, 0)))
    candt = jnp.pad(candp.T, ((0, 4), (0, 0)))
    srow = jnp.full((8, _C), -1e30, dtype=jnp.float32)
    srow = srow.at[0, :PRE_NMS_TOPK].set(cand_scores)

    kept = pl.pallas_call(
        _nms_kernel,
        out_shape=jax.ShapeDtypeStruct((8, _C), jnp.float32),
        scratch_shapes=[pltpu.VMEM((_C, _C), jnp.float32)],
    )(candp, candt, srow)
    kept_scores = kept[0, :PRE_NMS_TOPK]

    det_scores, det_idx = jax.lax.top_k(kept_scores, DET_PER_IMG)
    valid = det_scores > -1e29
    det_boxes = jnp.where(valid[:, None], cand_boxes[det_idx], 0.0)
    det_scores = jnp.where(valid, det_scores, 0.0)
    det = jnp.concatenate([det_boxes, det_scores[:, None]], axis=1)
    return det, matched_idxs, match_labels
